# baseline (speedup 1.0000x reference)
import jax
import jax.numpy as jnp
from jax.experimental import pallas as pl
from jax.experimental.pallas import tpu as pltpu

N_EXPERT = 16
DIM_IN = 2048
TILE = 1024
NSTREAM = 2


def _gate(logits, cw_ref, ei_ref, s):
    t = logits.shape[0]
    iota = jax.lax.broadcasted_iota(jnp.int32, (t, N_EXPERT), 1)

    m1 = jnp.max(logits, axis=-1, keepdims=True)
    idx1 = jnp.min(
        jnp.where(logits == m1, iota, N_EXPERT), axis=-1, keepdims=True
    )
    masked = jnp.where(iota == idx1, -jnp.inf, logits)
    m2 = jnp.max(masked, axis=-1, keepdims=True)
    idx2 = jnp.min(
        jnp.where(masked == m2, iota, N_EXPERT), axis=-1, keepdims=True
    )

    z = jnp.sum(jnp.exp(logits - m1), axis=-1, keepdims=True)
    p1 = 1.0 / z
    p2 = jnp.exp(m2 - m1) / z
    den = p1 + p2 + 1e-09
    cw_ref[s, :, 0:1] = p1 / den
    cw_ref[s, :, 1:2] = p2 / den
    ei_ref[s, :, 0:1] = idx1
    ei_ref[s, :, 1:2] = idx2


def _gating_kernel(*refs):
    wt_ref = refs[0]
    x_refs = refs[1 : 1 + NSTREAM]
    cw_ref, ei_ref = refs[1 + NSTREAM :]
    wt = wt_ref[...]
    for s in range(NSTREAM):
        logits = jax.lax.dot_general(
            x_refs[s][0], wt, (((1,), (0,)), ((), ())),
            preferred_element_type=jnp.float32,
        )
        _gate(logits, cw_ref, ei_ref, s)


def kernel(x, W):
    b, n, d = x.shape
    tokens = b * n
    rows = tokens // NSTREAM
    xf = x.reshape(NSTREAM, rows, d)
    wt = W.T
    grid = (rows // TILE,)
    x_specs = [
        pl.BlockSpec((1, TILE, d), lambda i, s=s: (s, i, 0))
        for s in range(NSTREAM)
    ]
    cw, ei = pl.pallas_call(
        _gating_kernel,
        grid=grid,
        in_specs=[pl.BlockSpec((d, N_EXPERT), lambda i: (0, 0))] + x_specs,
        out_specs=[
            pl.BlockSpec((NSTREAM, TILE, 2), lambda i: (0, i, 0)),
            pl.BlockSpec((NSTREAM, TILE, 2), lambda i: (0, i, 0)),
        ],
        out_shape=[
            jax.ShapeDtypeStruct((NSTREAM, rows, 2), jnp.float32),
            jax.ShapeDtypeStruct((NSTREAM, rows, 2), jnp.int32),
        ],
        compiler_params=pltpu.CompilerParams(
            dimension_semantics=("parallel",),
        ),
    )(wt, *([xf] * NSTREAM))
    return cw.reshape(b, n, 2), ei.reshape(b, n, 2)
